# baseline (speedup 1.0000x reference)
import functools

import jax
import jax.numpy as jnp
from jax import lax
from jax.experimental import pallas as pl
from jax.experimental.pallas import tpu as pltpu
from jax.experimental.pallas import tpu_sc as plsc

_LANES = 16
_B = 128
_CPW = 51
_SPAN = _B * _CPW
_NW = 32
_DEPTH = 3


def _body(x_hbm, upx_hbm, idx_hbm, out_hbm,
          idx_all, g0, g1, g2, u0, u1, u2,
          sg0, sg1, sg2, su0, su1, su2, so0, so1, so2, *, m, n_col):
    gath = (g0, g1, g2)
    upx = (u0, u1, u2)
    sg = (sg0, sg1, sg2)
    su = (su0, su1, su2)
    so = (so0, so1, so2)
    ngrp = n_col // _LANES

    wid = lax.axis_index("s") * 2 + lax.axis_index("c")
    pb = jnp.minimum(wid * _SPAN, m - _SPAN)
    pltpu.sync_copy(idx_hbm.at[pl.ds(pb, _SPAN)], idx_all)

    def in_copies(k, b):
        idx_slice = idx_all.at[pl.ds(k * _B, _B)]
        return (
            pltpu.make_async_copy(x_hbm.at[idx_slice], gath[b], sg[b]),
            pltpu.make_async_copy(
                upx_hbm.at[pl.ds(pb + k * _B, _B)], upx[b], su[b]),
        )

    def out_copy(k, b):
        return pltpu.make_async_copy(
            upx[b], out_hbm.at[pl.ds(pb + k * _B, _B)], so[b])

    for b in range(2):
        for cp in in_copies(b, b):
            cp.start()

    @pl.loop(0, _CPW, step=_DEPTH)
    def _ring(k0):
        for db in range(_DEPTH):
            b = db
            k = k0 + db
            for cp in in_copies(k, b):
                cp.wait()

            @plsc.parallel_loop(0, _B, unroll=2)
            def _row(r):
                g = [gath[b][r, pl.ds(j * _LANES, _LANES)]
                     for j in range(ngrp)]
                for j in range(ngrp):
                    plsc.addupdate(
                        upx[b].at[r, pl.ds(j * _LANES, _LANES)], g[j])

            out_copy(k, b).start()
            b2 = (db + 2) % _DEPTH

            @pl.when(k + 2 < _CPW)
            def _():
                @pl.when(k >= 1)
                def _():
                    out_copy(k - 1, b2).wait()

                for cp in in_copies(k + 2, b2):
                    cp.start()

    for j in range(_DEPTH):
        out_copy(_CPW - _DEPTH + j, j).wait()


def kernel(x, up_x, unq_inv):
    m, n_col = up_x.shape
    idx = unq_inv.astype(jnp.int32)
    assert _SPAN * _NW >= m and _SPAN <= m

    mesh = plsc.VectorSubcoreMesh(core_axis_name="c", subcore_axis_name="s")
    body = functools.partial(_body, m=m, n_col=n_col)
    run = pl.kernel(
        body,
        out_type=jax.ShapeDtypeStruct((m, n_col), jnp.float32),
        mesh=mesh,
        scratch_types=[
            pltpu.VMEM((_SPAN,), jnp.int32),
            pltpu.VMEM((_B, n_col), jnp.float32),
            pltpu.VMEM((_B, n_col), jnp.float32),
            pltpu.VMEM((_B, n_col), jnp.float32),
            pltpu.VMEM((_B, n_col), jnp.float32),
            pltpu.VMEM((_B, n_col), jnp.float32),
            pltpu.VMEM((_B, n_col), jnp.float32),
            pltpu.SemaphoreType.DMA,
            pltpu.SemaphoreType.DMA,
            pltpu.SemaphoreType.DMA,
            pltpu.SemaphoreType.DMA,
            pltpu.SemaphoreType.DMA,
            pltpu.SemaphoreType.DMA,
            pltpu.SemaphoreType.DMA,
            pltpu.SemaphoreType.DMA,
            pltpu.SemaphoreType.DMA,
        ],
    )
    return run(x, up_x, idx)


# speedup vs baseline: 1.0135x; 1.0135x over previous
import functools

import jax
import jax.numpy as jnp
from jax import lax
from jax.experimental import pallas as pl
from jax.experimental.pallas import tpu as pltpu
from jax.experimental.pallas import tpu_sc as plsc

_LANES = 16
_B = 64
_DEPTH = 6
_AHEAD = 4
_CPW = 102
_SPAN = _B * _CPW
_NW = 32


def _body(x_hbm, upx_hbm, idx_hbm, out_hbm, idx_all, *rest, m, n_col):
    gath = rest[:_DEPTH]
    upx = rest[_DEPTH:2 * _DEPTH]
    sg = rest[2 * _DEPTH:3 * _DEPTH]
    su = rest[3 * _DEPTH:4 * _DEPTH]
    so = rest[4 * _DEPTH:5 * _DEPTH]
    ngrp = n_col // _LANES

    wid = lax.axis_index("s") * 2 + lax.axis_index("c")
    pb = jnp.minimum(wid * _SPAN, m - _SPAN)
    pltpu.sync_copy(idx_hbm.at[pl.ds(pb, _SPAN)], idx_all)

    def in_copies(k, b):
        idx_slice = idx_all.at[pl.ds(k * _B, _B)]
        return (
            pltpu.make_async_copy(x_hbm.at[idx_slice], gath[b], sg[b]),
            pltpu.make_async_copy(
                upx_hbm.at[pl.ds(pb + k * _B, _B)], upx[b], su[b]),
        )

    def out_copy(k, b):
        return pltpu.make_async_copy(
            upx[b], out_hbm.at[pl.ds(pb + k * _B, _B)], so[b])

    for b in range(_AHEAD):
        for cp in in_copies(b, b):
            cp.start()

    @pl.loop(0, _CPW, step=_DEPTH)
    def _ring(k0):
        for db in range(_DEPTH):
            b = db
            k = k0 + db
            for cp in in_copies(k, b):
                cp.wait()

            @plsc.parallel_loop(0, _B, unroll=2)
            def _row(r):
                g = [gath[b][r, pl.ds(j * _LANES, _LANES)]
                     for j in range(ngrp)]
                for j in range(ngrp):
                    plsc.addupdate(
                        upx[b].at[r, pl.ds(j * _LANES, _LANES)], g[j])

            out_copy(k, b).start()
            b2 = (db + _AHEAD) % _DEPTH

            @pl.when(k + _AHEAD < _CPW)
            def _():
                @pl.when(k >= _DEPTH - _AHEAD)
                def _():
                    out_copy(k - (_DEPTH - _AHEAD), b2).wait()

                for cp in in_copies(k + _AHEAD, b2):
                    cp.start()

    for j in range(_DEPTH):
        k_last = _CPW - _DEPTH + j
        out_copy(k_last, k_last % _DEPTH).wait()


def kernel(x, up_x, unq_inv):
    m, n_col = up_x.shape
    idx = unq_inv.astype(jnp.int32)
    assert _SPAN * _NW >= m and _SPAN <= m

    mesh = plsc.VectorSubcoreMesh(core_axis_name="c", subcore_axis_name="s")
    body = functools.partial(_body, m=m, n_col=n_col)
    run = pl.kernel(
        body,
        out_type=jax.ShapeDtypeStruct((m, n_col), jnp.float32),
        mesh=mesh,
        scratch_types=(
            [pltpu.VMEM((_SPAN,), jnp.int32)]
            + [pltpu.VMEM((_B, n_col), jnp.float32)] * (2 * _DEPTH)
            + [pltpu.SemaphoreType.DMA] * (3 * _DEPTH)
        ),
    )
    return run(x, up_x, idx)


# speedup vs baseline: 1.0430x; 1.0291x over previous
import functools

import jax
import jax.numpy as jnp
from jax import lax
from jax.experimental import pallas as pl
from jax.experimental.pallas import tpu as pltpu
from jax.experimental.pallas import tpu_sc as plsc

_LANES = 16
_B = 64
_DEPTH = 7
_AHEAD = 5
_CPW = 98
_SPAN = _B * _CPW
_NW = 32


def _body(x_hbm, upx_hbm, idx_hbm, out_hbm, idx_all, *rest, m, n_col):
    gath = rest[:_DEPTH]
    upx = rest[_DEPTH:2 * _DEPTH]
    sg = rest[2 * _DEPTH:3 * _DEPTH]
    su = rest[3 * _DEPTH:4 * _DEPTH]
    so = rest[4 * _DEPTH:5 * _DEPTH]
    ngrp = n_col // _LANES

    wid = lax.axis_index("s") * 2 + lax.axis_index("c")
    pb = jnp.minimum(wid * _SPAN, m - _SPAN)
    pltpu.sync_copy(idx_hbm.at[pl.ds(pb, _SPAN)], idx_all)

    def in_copies(k, b):
        idx_slice = idx_all.at[pl.ds(k * _B, _B)]
        return (
            pltpu.make_async_copy(x_hbm.at[idx_slice], gath[b], sg[b]),
            pltpu.make_async_copy(
                upx_hbm.at[pl.ds(pb + k * _B, _B)], upx[b], su[b]),
        )

    def out_copy(k, b):
        return pltpu.make_async_copy(
            upx[b], out_hbm.at[pl.ds(pb + k * _B, _B)], so[b])

    for b in range(_AHEAD):
        for cp in in_copies(b, b):
            cp.start()

    @pl.loop(0, _CPW, step=_DEPTH)
    def _ring(k0):
        for db in range(_DEPTH):
            b = db
            k = k0 + db
            for cp in in_copies(k, b):
                cp.wait()

            @plsc.parallel_loop(0, _B, unroll=2)
            def _row(r):
                g = [gath[b][r, pl.ds(j * _LANES, _LANES)]
                     for j in range(ngrp)]
                for j in range(ngrp):
                    plsc.addupdate(
                        upx[b].at[r, pl.ds(j * _LANES, _LANES)], g[j])

            out_copy(k, b).start()
            b2 = (db + _AHEAD) % _DEPTH

            @pl.when(k + _AHEAD < _CPW)
            def _():
                @pl.when(k >= _DEPTH - _AHEAD)
                def _():
                    out_copy(k - (_DEPTH - _AHEAD), b2).wait()

                for cp in in_copies(k + _AHEAD, b2):
                    cp.start()

    for j in range(_DEPTH):
        k_last = _CPW - _DEPTH + j
        out_copy(k_last, k_last % _DEPTH).wait()


def kernel(x, up_x, unq_inv):
    m, n_col = up_x.shape
    idx = unq_inv.astype(jnp.int32)
    assert _SPAN * _NW >= m and _SPAN <= m

    mesh = plsc.VectorSubcoreMesh(core_axis_name="c", subcore_axis_name="s")
    body = functools.partial(_body, m=m, n_col=n_col)
    run = pl.kernel(
        body,
        out_type=jax.ShapeDtypeStruct((m, n_col), jnp.float32),
        mesh=mesh,
        scratch_types=(
            [pltpu.VMEM((_SPAN,), jnp.int32)]
            + [pltpu.VMEM((_B, n_col), jnp.float32)] * (2 * _DEPTH)
            + [pltpu.SemaphoreType.DMA] * (3 * _DEPTH)
        ),
    )
    return run(x, up_x, idx)


# speedup vs baseline: 1.0550x; 1.0115x over previous
import functools

import jax
import jax.numpy as jnp
from jax import lax
from jax.experimental import pallas as pl
from jax.experimental.pallas import tpu as pltpu
from jax.experimental.pallas import tpu_sc as plsc

_LANES = 16
_B = 64
_DEPTH = 7
_AHEAD = 5
_CPW = 98
_SPAN = _B * _CPW
_NW = 32


def _body(x_hbm, upx_hbm, idx_hbm, out_hbm, idx_all, *rest, m, n_col):
    gath = rest[:_DEPTH]
    upx = rest[_DEPTH:2 * _DEPTH]
    sg = rest[2 * _DEPTH:3 * _DEPTH]
    su = rest[3 * _DEPTH:4 * _DEPTH]
    so = rest[4 * _DEPTH:5 * _DEPTH]
    sg2 = rest[5 * _DEPTH:6 * _DEPTH]
    ngrp = n_col // _LANES

    wid = lax.axis_index("s") * 2 + lax.axis_index("c")
    pb = jnp.minimum(wid * _SPAN, m - _SPAN)
    pltpu.sync_copy(idx_hbm.at[pl.ds(pb, _SPAN)], idx_all)

    half = _B // 2

    def in_copies(k, b):
        idx_lo = idx_all.at[pl.ds(k * _B, half)]
        idx_hi = idx_all.at[pl.ds(k * _B + half, half)]
        return (
            pltpu.make_async_copy(
                x_hbm.at[idx_lo], gath[b].at[pl.ds(0, half)], sg[b]),
            pltpu.make_async_copy(
                x_hbm.at[idx_hi], gath[b].at[pl.ds(half, half)], sg2[b]),
            pltpu.make_async_copy(
                upx_hbm.at[pl.ds(pb + k * _B, _B)], upx[b], su[b]),
        )

    def out_copy(k, b):
        return pltpu.make_async_copy(
            upx[b], out_hbm.at[pl.ds(pb + k * _B, _B)], so[b])

    for b in range(_AHEAD):
        for cp in in_copies(b, b):
            cp.start()

    @pl.loop(0, _CPW, step=_DEPTH)
    def _ring(k0):
        for db in range(_DEPTH):
            b = db
            k = k0 + db
            for cp in in_copies(k, b):
                cp.wait()

            @plsc.parallel_loop(0, _B, unroll=2)
            def _row(r):
                g = [gath[b][r, pl.ds(j * _LANES, _LANES)]
                     for j in range(ngrp)]
                for j in range(ngrp):
                    plsc.addupdate(
                        upx[b].at[r, pl.ds(j * _LANES, _LANES)], g[j])

            out_copy(k, b).start()
            b2 = (db + _AHEAD) % _DEPTH

            @pl.when(k + _AHEAD < _CPW)
            def _():
                @pl.when(k >= _DEPTH - _AHEAD)
                def _():
                    out_copy(k - (_DEPTH - _AHEAD), b2).wait()

                for cp in in_copies(k + _AHEAD, b2):
                    cp.start()

    for j in range(_DEPTH):
        k_last = _CPW - _DEPTH + j
        out_copy(k_last, k_last % _DEPTH).wait()


def kernel(x, up_x, unq_inv):
    m, n_col = up_x.shape
    idx = unq_inv.astype(jnp.int32)
    assert _SPAN * _NW >= m and _SPAN <= m

    mesh = plsc.VectorSubcoreMesh(core_axis_name="c", subcore_axis_name="s")
    body = functools.partial(_body, m=m, n_col=n_col)
    run = pl.kernel(
        body,
        out_type=jax.ShapeDtypeStruct((m, n_col), jnp.float32),
        mesh=mesh,
        scratch_types=(
            [pltpu.VMEM((_SPAN,), jnp.int32)]
            + [pltpu.VMEM((_B, n_col), jnp.float32)] * (2 * _DEPTH)
            + [pltpu.SemaphoreType.DMA] * (4 * _DEPTH)
        ),
    )
    return run(x, up_x, idx)
